# baseline (speedup 1.0000x reference)
import jax
import jax.numpy as jnp
from jax import lax
from jax.experimental import pallas as pl
from jax.experimental.pallas import tpu as pltpu
from jax.experimental.pallas import tpu_sc as plsc

N = 16
D = 32
H = 64
B = 2048
ND = N * D
NL = 2 * N
NW = 32
PW = B // NW
LB = 16
TI = 4

_C00 = (((0,), (0,)), ((), ()))


def _sc_body(locs_hbm, states_hbm, lut_hbm, m_hbm, ds_hbm,
             sv, lv, ftv, ltv, atv, dvv, mv, dsv, lutv):
    wid = lax.axis_index("s") * 2 + lax.axis_index("c")
    pltpu.sync_copy(lut_hbm, lutv)
    bidx = jnp.arange(LB, dtype=jnp.int32)
    bs = bidx * ND
    bl = bidx * NL
    zero = jnp.zeros((LB,), jnp.float32)

    def group(g, carry):
        base = wid * PW + g * LB
        pltpu.sync_copy(states_hbm.at[pl.ds(base * ND, LB * ND)], sv)
        pltpu.sync_copy(locs_hbm.at[pl.ds(base * NL, LB * NL)], lv)

        def tr_j(j, c):
            for d in range(D):
                v = plsc.load_gather(sv, [bs + (j * D + d)])
                ftv[pl.ds((j * D + d) * LB, LB)] = v
            return c
        lax.fori_loop(0, N, tr_j, 0)
        for i in range(N):
            ltv[pl.ds(i * LB, LB)] = plsc.load_gather(lv, [bl + 2 * i])
            ltv[pl.ds((N + i) * LB, LB)] = plsc.load_gather(lv,
                                                            [bl + 2 * i + 1])

        def adj_i(i, c):
            lxi = ltv[pl.ds(i * LB, LB)]
            lyi = ltv[pl.ds((N + i) * LB, LB)]
            deg = zero
            for j in range(N):
                dx = lxi - ltv[pl.ds(j * LB, LB)]
                dy = lyi - ltv[pl.ds((N + j) * LB, LB)]
                a0 = jnp.where(dx * dx + dy * dy < 1.0, 1.0, 0.0)
                atv[pl.ds((i * N + j) * LB, LB)] = a0
                deg = deg + a0
            dvv[pl.ds(i * LB, LB)] = plsc.load_gather(
                lutv, [deg.astype(jnp.int32)])
            return c
        lax.fori_loop(0, N, adj_i, 0)

        def norm_i(i, c):
            di = dvv[pl.ds(i * LB, LB)]
            rs = zero
            ax = zero
            ay = zero
            for j in range(N):
                at = atv[pl.ds((i * N + j) * LB, LB)] * di
                at = at * dvv[pl.ds(j * LB, LB)]
                at = jnp.where(i == j, zero, at)
                atv[pl.ds((i * N + j) * LB, LB)] = at
                rs = rs + at
                ax = ax + at * ltv[pl.ds(j * LB, LB)]
                ay = ay + at * ltv[pl.ds((N + j) * LB, LB)]
            lxi = ltv[pl.ds(i * LB, LB)]
            lyi = ltv[pl.ds((N + i) * LB, LB)]
            plsc.store_scatter(dsv, [bl + i], rs * lxi - ax)
            plsc.store_scatter(dsv, [bl + (N + i)], rs * lyi - ay)
            return c
        lax.fori_loop(0, N, norm_i, 0)

        def agg_ib(ib, c):
            def agg_db(db, c2):
                acc = [[zero for _ in range(8)] for _ in range(TI)]
                for j in range(N):
                    fj = [ftv[pl.ds((j * D + db * 8 + k) * LB, LB)]
                          for k in range(8)]
                    for ti in range(TI):
                        at = atv[pl.ds(((ib * TI + ti) * N + j) * LB, LB)]
                        for k in range(8):
                            acc[ti][k] = acc[ti][k] + at * fj[k]
                for ti in range(TI):
                    for k in range(8):
                        col = (ib * TI + ti) * D + db * 8 + k
                        plsc.store_scatter(mv, [bs + col], acc[ti][k])
                return c2
            lax.fori_loop(0, D // 8, agg_db, 0)
            return c
        lax.fori_loop(0, N // TI, agg_ib, 0)

        pltpu.sync_copy(mv, m_hbm.at[pl.ds(base * ND, LB * ND)])
        pltpu.sync_copy(dsv, ds_hbm.at[pl.ds(base * NL, LB * NL)])
        return carry

    lax.fori_loop(0, PW // LB, group, 0)


def _sc_stage(locs_flat, states_flat, lut):
    mesh = plsc.VectorSubcoreMesh(core_axis_name="c", subcore_axis_name="s")
    kern = pl.kernel(
        _sc_body,
        mesh=mesh,
        compiler_params=pltpu.CompilerParams(needs_layout_passes=False),
        out_type=[
            jax.ShapeDtypeStruct((B * ND,), jnp.float32),
            jax.ShapeDtypeStruct((B * NL,), jnp.float32),
        ],
        scratch_types=[
            pltpu.VMEM((LB * ND,), jnp.float32),
            pltpu.VMEM((LB * NL,), jnp.float32),
            pltpu.VMEM((ND * LB,), jnp.float32),
            pltpu.VMEM((NL * LB,), jnp.float32),
            pltpu.VMEM((N * N * LB,), jnp.float32),
            pltpu.VMEM((N * LB,), jnp.float32),
            pltpu.VMEM((LB * ND,), jnp.float32),
            pltpu.VMEM((LB * NL,), jnp.float32),
            pltpu.VMEM((D,), jnp.float32),
        ],
    )
    return kern(locs_flat, states_flat, lut)


def _tc_body(states_ref, m_ref, ds_ref, w1_ref, b1_ref, w4_ref, w5_ref,
             b4_ref, b5_ref, pol_ref, val_ref):
    ft = states_ref[...].T
    mt = m_ref[...].T
    dst = ds_ref[...].T

    w1 = w1_ref[...]
    w1a15 = w1[:D] * (N - 1.0)
    w1b = w1[D:2 * D]
    w1cb = jnp.concatenate([w1[2 * D:], (N - 1.0) * b1_ref[...]], axis=0)
    w45b = jnp.concatenate([
        jnp.concatenate([w4_ref[...], w5_ref[...]], axis=1),
        jnp.concatenate([b4_ref[...], b5_ref[...]], axis=1),
    ], axis=0)
    ones = jnp.ones((1, B), jnp.float32)

    rows_p = []
    rows_v = []
    for i in range(N):
        x = lax.dot_general(w1a15, ft[i * D:(i + 1) * D], _C00,
                            preferred_element_type=jnp.float32)
        x = x + lax.dot_general(w1b, mt[i * D:(i + 1) * D], _C00,
                                preferred_element_type=jnp.float32)
        dse = jnp.concatenate([dst[i:i + 1], dst[N + i:N + i + 1], ones],
                              axis=0)
        x = x + lax.dot_general(w1cb, dse, _C00,
                                preferred_element_type=jnp.float32)
        s2 = jnp.maximum(x, 0.0)
        s2e = jnp.concatenate([s2, ones], axis=0)
        pv = lax.dot_general(w45b, s2e, _C00,
                             preferred_element_type=jnp.float32)
        rows_p.append(pv[0:2])
        rows_v.append(pv[2:3])

    pol_ref[...] = jnp.concatenate(rows_p, axis=0).T
    val_ref[...] = jnp.concatenate(rows_v, axis=0).T


@jax.jit
def kernel(locs, states, W1, b1, W4, b4, W5, b5):
    lut = 1.0 / jnp.sqrt(jnp.arange(D, dtype=jnp.float32).clip(1.0))

    m_flat, ds_flat = _sc_stage(locs.reshape(B * NL), states.reshape(B * ND),
                                lut)

    pol, val = pl.pallas_call(
        _tc_body,
        in_specs=[
            pl.BlockSpec((B, ND), lambda: (0, 0)),
            pl.BlockSpec((B, ND), lambda: (0, 0)),
            pl.BlockSpec((B, NL), lambda: (0, 0)),
            pl.BlockSpec((2 * D + 2, H), lambda: (0, 0)),
            pl.BlockSpec((1, H), lambda: (0, 0)),
            pl.BlockSpec((H, 2), lambda: (0, 0)),
            pl.BlockSpec((H, 1), lambda: (0, 0)),
            pl.BlockSpec((1, 2), lambda: (0, 0)),
            pl.BlockSpec((1, 1), lambda: (0, 0)),
        ],
        out_specs=[
            pl.BlockSpec((B, N * 2), lambda: (0, 0)),
            pl.BlockSpec((B, N), lambda: (0, 0)),
        ],
        out_shape=[
            jax.ShapeDtypeStruct((B, N * 2), jnp.float32),
            jax.ShapeDtypeStruct((B, N), jnp.float32),
        ],
    )(states.reshape(B, ND), m_flat.reshape(B, ND), ds_flat.reshape(B, NL),
      W1, b1[None, :], W4, W5, b4[None, :], b5[None, :])

    return pol.reshape(B, N, 2), val.reshape(B, N, 1)


# speedup vs baseline: 5.3759x; 5.3759x over previous
import jax
import jax.numpy as jnp
from jax import lax
from jax.experimental import pallas as pl
from jax.experimental.pallas import tpu as pltpu

N = 16
D = 32
H = 64
BB = 2048

_C00 = (((0,), (0,)), ((), ()))


def _body(locs_ref, states_ref, w1_ref, b1_ref, w4_ref, w5_ref, b4_ref,
          b5_ref, pol_ref, val_ref, a0_ref):
    ft = states_ref[...].T
    lti = locs_ref[...].T

    r = lax.broadcasted_iota(jnp.int32, (2 * N, 2 * N), 0)
    s = lax.broadcasted_iota(jnp.int32, (2 * N, 2 * N), 1)
    perm = (s == 2 * (r % N) + r // N).astype(jnp.float32)
    lt = jnp.dot(perm, lti, preferred_element_type=jnp.float32)
    lx = lt[:N]
    ly = lt[N:]

    degs = []
    for i in range(N):
        dx = lx[i:i + 1] - lx
        dy = ly[i:i + 1] - ly
        a0row = ((dx * dx + dy * dy) < 1.0).astype(jnp.float32)
        a0_ref[i] = a0row
        degs.append(jnp.sum(a0row, axis=0, keepdims=True))
    dinv = lax.rsqrt(jnp.concatenate(degs, axis=0))

    gs = [ft[j * D:(j + 1) * D] * dinv[j:j + 1] for j in range(N)]
    glx = lx * dinv
    gly = ly * dinv

    w1 = w1_ref[...]
    w1a15 = w1[:D] * (N - 1.0)
    w1b = w1[D:2 * D]
    w1cb = jnp.concatenate([w1[2 * D:], (N - 1.0) * b1_ref[...]], axis=0)
    w45b = jnp.concatenate([
        jnp.concatenate([w4_ref[...], w5_ref[...]], axis=1),
        jnp.concatenate([b4_ref[...], b5_ref[...]], axis=1),
    ], axis=0)
    ones = jnp.ones((1, BB), jnp.float32)

    rows_p = []
    rows_v = []
    for i in range(N):
        a0row = a0_ref[i]
        di = dinv[i:i + 1]
        agg = a0row[0:1] * gs[0]
        for j in range(1, N):
            agg = agg + a0row[j:j + 1] * gs[j]
        acc = di * (agg - gs[i])

        t = jnp.sum(a0row * dinv, axis=0, keepdims=True)
        rs = di * t - di * di
        sx = jnp.sum(a0row * glx, axis=0, keepdims=True)
        sy = jnp.sum(a0row * gly, axis=0, keepdims=True)
        dsx = rs * lx[i:i + 1] - di * (sx - glx[i:i + 1])
        dsy = rs * ly[i:i + 1] - di * (sy - gly[i:i + 1])

        x = lax.dot_general(w1a15, ft[i * D:(i + 1) * D], _C00,
                            preferred_element_type=jnp.float32)
        x = x + lax.dot_general(w1b, acc, _C00,
                                preferred_element_type=jnp.float32)
        dse = jnp.concatenate([dsx, dsy, ones], axis=0)
        x = x + lax.dot_general(w1cb, dse, _C00,
                                preferred_element_type=jnp.float32)
        s2 = jnp.maximum(x, 0.0)
        s2e = jnp.concatenate([s2, ones], axis=0)
        pv = lax.dot_general(w45b, s2e, _C00,
                             preferred_element_type=jnp.float32)
        rows_p.append(pv[0:2])
        rows_v.append(pv[2:3])

    pol_ref[...] = jnp.concatenate(rows_p, axis=0).T
    val_ref[...] = jnp.concatenate(rows_v, axis=0).T


@jax.jit
def kernel(locs, states, W1, b1, W4, b4, W5, b5):
    B = locs.shape[0]
    G = B // BB

    pol, val = pl.pallas_call(
        _body,
        grid=(G,),
        in_specs=[
            pl.BlockSpec((BB, 2 * N), lambda g: (g, 0)),
            pl.BlockSpec((BB, N * D), lambda g: (g, 0)),
            pl.BlockSpec((2 * D + 2, H), lambda g: (0, 0)),
            pl.BlockSpec((1, H), lambda g: (0, 0)),
            pl.BlockSpec((H, 2), lambda g: (0, 0)),
            pl.BlockSpec((H, 1), lambda g: (0, 0)),
            pl.BlockSpec((1, 2), lambda g: (0, 0)),
            pl.BlockSpec((1, 1), lambda g: (0, 0)),
        ],
        out_specs=[
            pl.BlockSpec((BB, N * 2), lambda g: (g, 0)),
            pl.BlockSpec((BB, N), lambda g: (g, 0)),
        ],
        out_shape=[
            jax.ShapeDtypeStruct((B, N * 2), jnp.float32),
            jax.ShapeDtypeStruct((B, N), jnp.float32),
        ],
        scratch_shapes=[pltpu.VMEM((N, N, BB), jnp.float32)],
    )(locs.reshape(B, 2 * N), states.reshape(B, N * D), W1, b1[None, :],
      W4, W5, b4[None, :], b5[None, :])

    return pol.reshape(B, N, 2), val.reshape(B, N, 1)
